# baseline (speedup 1.0000x reference)
import functools

import jax
import jax.numpy as jnp
from jax import lax
from jax.experimental import pallas as pl
from jax.experimental.pallas import tpu as pltpu
from jax.experimental.pallas import tpu_sc as plsc

_BEAM = 16
_NEG = float("-inf")


def _topk16_body(x_ref, vals_ref, idx_ref):
    x = x_ref[...]
    a = x.shape[1]
    m = jnp.max(x, axis=1, keepdims=True)
    s = jnp.sum(jnp.exp(x - m), axis=1, keepdims=True)
    lse = m + jnp.log(s)
    iota = lax.broadcasted_iota(jnp.int32, x.shape, 1)
    cur = x
    for k in range(_BEAM):
        v = jnp.max(cur, axis=1, keepdims=True)
        hit = cur == v
        idx = jnp.min(jnp.where(hit, iota, a), axis=1, keepdims=True)
        vals_ref[:, k : k + 1] = v - lse
        idx_ref[:, k : k + 1] = idx
        cur = jnp.where(iota == idx, _NEG, cur)


def _row_topk16(x, block_rows):
    rows, a = x.shape
    return pl.pallas_call(
        _topk16_body,
        grid=(rows // block_rows,),
        in_specs=[pl.BlockSpec((block_rows, a), lambda i: (i, 0))],
        out_specs=[
            pl.BlockSpec((block_rows, _BEAM), lambda i: (i, 0)),
            pl.BlockSpec((block_rows, _BEAM), lambda i: (i, 0)),
        ],
        out_shape=[
            jax.ShapeDtypeStruct((rows, _BEAM), jnp.float32),
            jax.ShapeDtypeStruct((rows, _BEAM), jnp.int32),
        ],
    )(x)


def _merge_body(rep_ref, vals_ref, idx_ref, probs_ref, off_ref, fr3_ref):
    cand = rep_ref[...] + vals_ref[...]
    shape = cand.shape
    p = lax.broadcasted_iota(jnp.int32, shape, 1)
    g = (p // _BEAM) * 2048 + idx_ref[...]
    b_col = lax.broadcasted_iota(jnp.int32, (shape[0], 1), 0)
    iota3 = lax.broadcasted_iota(jnp.int32, (shape[0], 3), 1)
    big = jnp.int32(1 << 30)
    for k in range(_BEAM):
        v = jnp.max(cand, axis=1, keepdims=True)
        hit = cand == v
        gm = jnp.min(jnp.where(hit, g, big), axis=1, keepdims=True)
        probs_ref[:, k : k + 1] = v
        off_ref[:, k : k + 1] = gm // 2048
        fr3_ref[:, 3 * k : 3 * k + 3] = (b_col * (_BEAM * 2048) + gm) * 3 + iota3
        cand = jnp.where(g == gm, _NEG, cand)


def _merge(rep, vals1_r, idx1_r):
    b = rep.shape[0]
    spec = pl.BlockSpec(rep.shape, lambda: (0, 0))
    out_spec = pl.BlockSpec((b, _BEAM), lambda: (0, 0))
    return pl.pallas_call(
        _merge_body,
        in_specs=[spec, spec, spec],
        out_specs=[out_spec, out_spec, pl.BlockSpec((b, 3 * _BEAM), lambda: (0, 0))],
        out_shape=[
            jax.ShapeDtypeStruct((b, _BEAM), jnp.float32),
            jax.ShapeDtypeStruct((b, _BEAM), jnp.int32),
            jax.ShapeDtypeStruct((b, 3 * _BEAM), jnp.int32),
        ],
    )(rep, vals1_r, idx1_r)


def _sc_gather(table, flat_idx):
    info = plsc.get_sparse_core_info()
    nw = info.num_cores * info.num_subcores
    b = flat_idx.shape[0]
    b_per_w = b // nw
    mesh = plsc.VectorSubcoreMesh(core_axis_name="c", subcore_axis_name="s")

    @functools.partial(
        pl.kernel,
        mesh=mesh,
        out_type=jax.ShapeDtypeStruct((b,), jnp.int32),
        scratch_types=[
            pltpu.VMEM((b_per_w,), jnp.int32),
            pltpu.VMEM((b_per_w,), jnp.int32),
            pltpu.SemaphoreType.DMA,
        ],
    )
    def gk(idx_hbm, table_hbm, out_hbm, idx_v, rows_v, sem):
        wid = lax.axis_index("s") * info.num_cores + lax.axis_index("c")
        base = wid * b_per_w
        pltpu.sync_copy(idx_hbm.at[pl.ds(base, b_per_w)], idx_v)
        pltpu.async_copy(table_hbm.at[idx_v], rows_v, sem).wait()
        pltpu.sync_copy(rows_v, out_hbm.at[pl.ds(base, b_per_w)])

    return gk(flat_idx, table)


def kernel(logits0, action_space0, logits1, action_space1):
    del action_space0
    b, a = logits0.shape

    beam_lp, _ = _row_topk16(logits0, b)
    vals1, idx1 = _row_topk16(logits1, 128)

    rep = jnp.repeat(beam_lp, _BEAM, axis=1)
    probs, offset, fr3 = _merge(rep, vals1.reshape(b, -1), idx1.reshape(b, -1))

    table = action_space1.reshape(-1)
    rows = _sc_gather(table, fr3.reshape(-1)).reshape(-1, 3)
    rels = rows[:, 0].reshape(b, _BEAM)
    ents = rows[:, 1].reshape(b, _BEAM)
    ts = rows[:, 2].reshape(b, _BEAM)
    return (ents, probs, ts, rels, offset)


# speedup vs baseline: 1.1200x; 1.1200x over previous
import functools

import jax
import jax.numpy as jnp
from jax import lax
from jax.experimental import pallas as pl
from jax.experimental.pallas import tpu as pltpu

_BEAM = 16
_NEG = float("-inf")


def _topk16_body(x_ref, vals_ref, idx_ref):
    x = x_ref[...]
    a = x.shape[1]
    m = jnp.max(x, axis=1, keepdims=True)
    s = jnp.sum(jnp.exp(x - m), axis=1, keepdims=True)
    lse = m + jnp.log(s)
    iota = lax.broadcasted_iota(jnp.int32, x.shape, 1)
    cur = x
    for k in range(_BEAM):
        v = jnp.max(cur, axis=1, keepdims=True)
        hit = cur == v
        idx = jnp.min(jnp.where(hit, iota, a), axis=1, keepdims=True)
        vals_ref[:, k : k + 1] = v - lse
        idx_ref[:, k : k + 1] = idx
        cur = jnp.where(iota == idx, _NEG, cur)


def _row_topk16(x, block_rows):
    rows, a = x.shape
    return pl.pallas_call(
        _topk16_body,
        grid=(rows // block_rows,),
        in_specs=[pl.BlockSpec((block_rows, a), lambda i: (i, 0))],
        out_specs=[
            pl.BlockSpec((block_rows, _BEAM), lambda i: (i, 0)),
            pl.BlockSpec((block_rows, _BEAM), lambda i: (i, 0)),
        ],
        out_shape=[
            jax.ShapeDtypeStruct((rows, _BEAM), jnp.float32),
            jax.ShapeDtypeStruct((rows, _BEAM), jnp.int32),
        ],
    )(x)


def _merge_body(rep_ref, vals_ref, idx_ref, probs_ref, off_ref, frd_ref, frm_ref):
    cand = rep_ref[...] + vals_ref[...]
    shape = cand.shape
    p = lax.broadcasted_iota(jnp.int32, shape, 1)
    g = (p // _BEAM) * 2048 + idx_ref[...]
    b_col = lax.broadcasted_iota(jnp.int32, (shape[0], 1), 0)
    big = jnp.int32(1 << 30)
    for k in range(_BEAM):
        v = jnp.max(cand, axis=1, keepdims=True)
        hit = cand == v
        gm = jnp.min(jnp.where(hit, g, big), axis=1, keepdims=True)
        probs_ref[:, k : k + 1] = v
        off_ref[:, k : k + 1] = gm // 2048
        fr = b_col * (_BEAM * 2048) + gm
        frd_ref[:, k : k + 1] = fr // 8
        frm_ref[:, k : k + 1] = fr % 8
        cand = jnp.where(g == gm, _NEG, cand)


def _merge(rep, vals1_r, idx1_r):
    b = rep.shape[0]
    spec = pl.BlockSpec(rep.shape, lambda: (0, 0))
    out_spec = pl.BlockSpec((b, _BEAM), lambda: (0, 0))
    return pl.pallas_call(
        _merge_body,
        in_specs=[spec, spec, spec],
        out_specs=[out_spec, out_spec, out_spec, out_spec],
        out_shape=[
            jax.ShapeDtypeStruct((b, _BEAM), jnp.float32),
            jax.ShapeDtypeStruct((b, _BEAM), jnp.int32),
            jax.ShapeDtypeStruct((b, _BEAM), jnp.int32),
            jax.ShapeDtypeStruct((b, _BEAM), jnp.int32),
        ],
    )(rep, vals1_r, idx1_r)


def _gather_body(d8_ref, m8_ref, *refs):
    del d8_ref
    ins = refs[:_BEAM]
    out_ref = refs[_BEAM]
    b = pl.program_id(0)
    iota0 = lax.broadcasted_iota(jnp.int32, (8, 3), 0)
    parts = []
    for k in range(_BEAM):
        rm = m8_ref[b * _BEAM + k]
        blk = ins[k][...]
        tri = jnp.sum(jnp.where(iota0 == rm, blk, 0), axis=0)
        parts.append(tri)
    out_ref[...] = jnp.concatenate(parts, 0).reshape(1, 1, 3 * _BEAM)


def _tc_gather(table, div8, mod8):
    b = div8.shape[0] // _BEAM

    def mk_spec(k):
        return pl.BlockSpec((8, 3), lambda i, d8, m8, k=k: (d8[i * _BEAM + k], 0))

    grid_spec = pltpu.PrefetchScalarGridSpec(
        num_scalar_prefetch=2,
        grid=(b,),
        in_specs=[mk_spec(k) for k in range(_BEAM)],
        out_specs=pl.BlockSpec((1, 1, 3 * _BEAM), lambda i, d8, m8: (i, 0, 0)),
    )
    out = pl.pallas_call(
        _gather_body,
        grid_spec=grid_spec,
        out_shape=jax.ShapeDtypeStruct((b, 1, 3 * _BEAM), jnp.int32),
    )(div8, mod8, *([table] * _BEAM))
    return out.reshape(b, 3 * _BEAM)


def kernel(logits0, action_space0, logits1, action_space1):
    del action_space0
    b, a = logits0.shape

    beam_lp, _ = _row_topk16(logits0, b)
    vals1, idx1 = _row_topk16(logits1, 128)

    rep = jnp.repeat(beam_lp, _BEAM, axis=1)
    probs, offset, frd, frm = _merge(rep, vals1.reshape(b, -1), idx1.reshape(b, -1))

    table = action_space1.reshape(-1, 3)
    rows = _tc_gather(table, frd.reshape(-1), frm.reshape(-1))
    rels = rows[:, 0::3]
    ents = rows[:, 1::3]
    ts = rows[:, 2::3]
    return (ents, probs, ts, rels, offset)


# speedup vs baseline: 54.5952x; 48.7477x over previous
import functools

import jax
import jax.numpy as jnp
from jax import lax
from jax.experimental import pallas as pl
from jax.experimental.pallas import tpu as pltpu

_BEAM = 16
_NEG = float("-inf")


def _topk16_body(x_ref, vals_ref, idx_ref):
    x = x_ref[...]
    a = x.shape[1]
    m = jnp.max(x, axis=1, keepdims=True)
    s = jnp.sum(jnp.exp(x - m), axis=1, keepdims=True)
    lse = m + jnp.log(s)
    iota = lax.broadcasted_iota(jnp.int32, x.shape, 1)
    cur = x
    for k in range(_BEAM):
        v = jnp.max(cur, axis=1, keepdims=True)
        hit = cur == v
        idx = jnp.min(jnp.where(hit, iota, a), axis=1, keepdims=True)
        vals_ref[:, k : k + 1] = v - lse
        idx_ref[:, k : k + 1] = idx
        cur = jnp.where(iota == idx, _NEG, cur)


def _row_topk16(x, block_rows):
    rows, a = x.shape
    return pl.pallas_call(
        _topk16_body,
        grid=(rows // block_rows,),
        in_specs=[pl.BlockSpec((block_rows, a), lambda i: (i, 0))],
        out_specs=[
            pl.BlockSpec((block_rows, _BEAM), lambda i: (i, 0)),
            pl.BlockSpec((block_rows, _BEAM), lambda i: (i, 0)),
        ],
        out_shape=[
            jax.ShapeDtypeStruct((rows, _BEAM), jnp.float32),
            jax.ShapeDtypeStruct((rows, _BEAM), jnp.int32),
        ],
    )(x)


def _merge_body(rep_ref, vals_ref, idx_ref, probs_ref, off_ref,
                rq_ref, rs_ref, aq_ref, as_ref):
    cand = rep_ref[...] + vals_ref[...]
    shape = cand.shape
    p = lax.broadcasted_iota(jnp.int32, shape, 1)
    g = (p // _BEAM) * 2048 + idx_ref[...]
    b_col = lax.broadcasted_iota(jnp.int32, (shape[0], 1), 0)
    big = jnp.int32(1 << 30)
    for k in range(_BEAM):
        v = jnp.max(cand, axis=1, keepdims=True)
        hit = cand == v
        gm = jnp.min(jnp.where(hit, g, big), axis=1, keepdims=True)
        probs_ref[:, k : k + 1] = v
        j = gm // 2048
        a = gm - j * 2048
        off_ref[:, k : k + 1] = j
        r = b_col * _BEAM + j
        rq_ref[:, k : k + 1] = r // 8
        rs_ref[:, k : k + 1] = r % 8
        aq_ref[:, k : k + 1] = a // 128
        as_ref[:, k : k + 1] = a % 128
        cand = jnp.where(g == gm, _NEG, cand)


def _merge(rep, vals1_r, idx1_r):
    b = rep.shape[0]
    spec = pl.BlockSpec(rep.shape, lambda: (0, 0))
    out_spec = pl.BlockSpec((b, _BEAM), lambda: (0, 0))
    return pl.pallas_call(
        _merge_body,
        in_specs=[spec, spec, spec],
        out_specs=[out_spec] * 6,
        out_shape=[
            jax.ShapeDtypeStruct((b, _BEAM), jnp.float32),
            jax.ShapeDtypeStruct((b, _BEAM), jnp.int32),
            jax.ShapeDtypeStruct((b, _BEAM), jnp.int32),
            jax.ShapeDtypeStruct((b, _BEAM), jnp.int32),
            jax.ShapeDtypeStruct((b, _BEAM), jnp.int32),
            jax.ShapeDtypeStruct((b, _BEAM), jnp.int32),
        ],
    )(rep, vals1_r, idx1_r)


def _gather_body(rq_ref, aq_ref, rs_ref, as_ref, *refs):
    del rq_ref, aq_ref
    ins = refs[:_BEAM]
    out_ref = refs[_BEAM]
    b = pl.program_id(0)
    iota_s = lax.broadcasted_iota(jnp.int32, (3, 8, 128), 1)
    iota_l = lax.broadcasted_iota(jnp.int32, (3, 8, 128), 2)
    parts = []
    for k in range(_BEAM):
        rm = rs_ref[b * _BEAM + k]
        am = as_ref[b * _BEAM + k]
        blk = ins[k][...]
        hit = (iota_s == rm) & (iota_l == am)
        tri = jnp.sum(jnp.where(hit, blk, 0), axis=(1, 2))
        parts.append(tri)
    out_ref[...] = jnp.concatenate(parts, 0).reshape(1, 1, 3 * _BEAM)


def _tc_gather(planes, rq, rs, aq, asub):
    b = rq.shape[0] // _BEAM

    def mk_spec(k):
        return pl.BlockSpec(
            (3, 8, 128),
            lambda i, rq_r, aq_r, rs_r, as_r, k=k: (
                0,
                rq_r[i * _BEAM + k],
                aq_r[i * _BEAM + k],
            ),
        )

    grid_spec = pltpu.PrefetchScalarGridSpec(
        num_scalar_prefetch=4,
        grid=(b,),
        in_specs=[mk_spec(k) for k in range(_BEAM)],
        out_specs=pl.BlockSpec(
            (1, 1, 3 * _BEAM), lambda i, rq_r, aq_r, rs_r, as_r: (i, 0, 0)
        ),
    )
    out = pl.pallas_call(
        _gather_body,
        grid_spec=grid_spec,
        out_shape=jax.ShapeDtypeStruct((b, 1, 3 * _BEAM), jnp.int32),
    )(rq, aq, rs, asub, *([planes] * _BEAM))
    return out.reshape(b, 3 * _BEAM)


def kernel(logits0, action_space0, logits1, action_space1):
    del action_space0
    b, a = logits0.shape

    beam_lp, _ = _row_topk16(logits0, b)
    vals1, idx1 = _row_topk16(logits1, 128)

    rep = jnp.repeat(beam_lp, _BEAM, axis=1)
    probs, offset, rq, rs, aq, asub = _merge(
        rep, vals1.reshape(b, -1), idx1.reshape(b, -1)
    )

    planes = jnp.transpose(action_space1, (2, 0, 1))
    rows = _tc_gather(
        planes, rq.reshape(-1), rs.reshape(-1), aq.reshape(-1), asub.reshape(-1)
    )
    rels = rows[:, 0::3]
    ents = rows[:, 1::3]
    ts = rows[:, 2::3]
    return (ents, probs, ts, rels, offset)
